# baseline (speedup 1.0000x reference)
import functools

import jax
import jax.numpy as jnp
from jax import lax
from jax.experimental import pallas as pl
from jax.experimental.pallas import tpu as pltpu
from jax.experimental.pallas import tpu_sc as plsc

N = 10000
E = 320000
NG = 10
DIN = 10
DH = 300
DOUT = 31

DHH = 150
DHP = 160

R = 1000
NRB = N // R

CHUNK = 88
NTILES = 16
KPT = 228
NCHUNKS = NTILES * KPT
EPAD = NCHUNKS * CHUNK
NPAD = N + 16
ZROWS = 400
NZ = N // ZROWS

_f32 = jnp.float32


def _lift_msg_body(x_ref, wl_ref, bl_ref, wa_ref, ba_ref, wb_ref, bb_ref,
                   ma_ref, mb_ref):
    i = pl.program_id(0)
    h = jnp.dot(x_ref[pl.ds(i * R, R), :], wl_ref[...],
                preferred_element_type=_f32)
    h = h + bl_ref[...]
    ma = jnp.dot(h, wa_ref[...], preferred_element_type=_f32) + ba_ref[...]
    mb = jnp.dot(h, wb_ref[...], preferred_element_type=_f32) + bb_ref[...]
    ma_ref[...] = jnp.maximum(ma, 0.0)
    mb_ref[...] = jnp.maximum(mb, 0.0)


def _mid_body(aa_ref, ab_ref, wfa_ref, wfb_ref, bf_ref,
              wca_ref, bca_ref, wcb_ref, bcb_ref, ma_ref, mb_ref):
    h = (jnp.dot(aa_ref[...], wfa_ref[...], preferred_element_type=_f32)
         + jnp.dot(ab_ref[...], wfb_ref[...], preferred_element_type=_f32)
         + bf_ref[...])
    h = jnp.maximum(h, 0.0)
    ma = jnp.dot(h, wca_ref[...], preferred_element_type=_f32) + bca_ref[...]
    mb = jnp.dot(h, wcb_ref[...], preferred_element_type=_f32) + bcb_ref[...]
    ma_ref[...] = jnp.maximum(ma, 0.0)
    mb_ref[...] = jnp.maximum(mb, 0.0)


def _readout_body(aa_ref, ab_ref, wfa_ref, wfb_ref, bf_ref,
                  wro_ref, bro_ref, gid_ref, out_ref):
    i = pl.program_id(0)
    h = (jnp.dot(aa_ref[...], wfa_ref[...], preferred_element_type=_f32)
         + jnp.dot(ab_ref[...], wfb_ref[...], preferred_element_type=_f32)
         + bf_ref[...])
    h = jnp.maximum(h, 0.0)
    nl = jnp.dot(h, wro_ref[...], preferred_element_type=_f32) + bro_ref[...]
    gid = gid_ref[pl.ds(i * R, R), :]
    iota = lax.broadcasted_iota(jnp.int32, (R, NG), 1)
    oh = (gid == iota).astype(_f32)
    contrib = lax.dot_general(oh, nl, (((0,), (0,)), ((), ())),
                              preferred_element_type=_f32)

    @pl.when(i == 0)
    def _():
        out_ref[...] = jnp.zeros_like(out_ref)

    out_ref[...] += contrib


def _full(shape):
    return pl.BlockSpec(shape, lambda i: (0,) * len(shape))


def _rows(w):
    return pl.BlockSpec((R, w), lambda i: (i, 0))


_lift_msg = pl.pallas_call(
    _lift_msg_body,
    grid=(NRB,),
    in_specs=[
        _full((N, DIN)),
        _full((DIN, DH)), _full((1, DH)),
        _full((DH, DHP)), _full((1, DHP)),
        _full((DH, DHP)), _full((1, DHP)),
    ],
    out_specs=[_rows(DHP), _rows(DHP)],
    out_shape=[jax.ShapeDtypeStruct((N, DHP), _f32)] * 2,
)

_mid = pl.pallas_call(
    _mid_body,
    grid=(NRB,),
    in_specs=[
        _rows(DHP), _rows(DHP),
        _full((DHP, DH)), _full((DHP, DH)), _full((1, DH)),
        _full((DH, DHP)), _full((1, DHP)),
        _full((DH, DHP)), _full((1, DHP)),
    ],
    out_specs=[_rows(DHP), _rows(DHP)],
    out_shape=[jax.ShapeDtypeStruct((N, DHP), _f32)] * 2,
)

_readout = pl.pallas_call(
    _readout_body,
    grid=(NRB,),
    in_specs=[
        _rows(DHP), _rows(DHP),
        _full((DHP, DH)), _full((DHP, DH)), _full((1, DH)),
        _full((DH, DOUT)), _full((1, DOUT)),
        _full((N, 1)),
    ],
    out_specs=_full((NG, DOUT)),
    out_shape=jax.ShapeDtypeStruct((NG, DOUT), _f32),
)


def _sc_body(msga, msgb, epairs, zr, agga, aggb,
             sd0, sd1, sd2, sd3, rows0, rows1, aggsh,
             gsem0, gsem1, ssem0, ssem1, isem0, isem1, isem2, isem3):
    c = lax.axis_index("c")
    s = lax.axis_index("s")

    nz = (NZ - 1 - s) // NTILES + 1

    def zbody(i, carry):
        off = (s + i * NTILES) * ZROWS
        pltpu.sync_copy(zr, aggsh.at[pl.ds(off, ZROWS)])
        return carry

    lax.fori_loop(0, nz, zbody, 0)

    plsc.subcore_barrier()

    base = s * KPT

    sds = (sd0, sd1, sd2, sd3)
    isems = (isem0, isem1, isem2, isem3)
    rows = (rows0, rows1)
    gsems = (gsem0, gsem1)
    ssems = (ssem0, ssem1)
    NQ = KPT // 4

    def run_edges(msg_ref):
        def idx_start(k, sset):
            return pltpu.async_copy(epairs.at[base + k], sds[sset],
                                    isems[sset])

        def idx_wait(k, sset):
            pltpu.make_async_copy(epairs.at[base + k], sds[sset],
                                  isems[sset]).wait()

        def sadd_wait(b, sset):
            pltpu.make_async_copy(rows[b], aggsh.at[sds[sset].at[1]],
                                  ssems[b]).wait()

        idx_start(0, 0)
        idx_start(1, 1)

        def ebody(q, carry):
            k = 4 * q
            g = [None, None]
            for h in range(2):
                for j in range(2):
                    m = k + 2 * h + j
                    b = j
                    cs = 2 * h + j
                    ps = (cs + 2) % 4

                    if h == 0:
                        @pl.when(q > 0)
                        def _():
                            sadd_wait(b, ps)

                        idx_start(m + 2, ps)
                    else:
                        sadd_wait(b, ps)

                        @pl.when(q < NQ - 1)
                        def _():
                            idx_start(m + 2, ps)

                    idx_wait(m, cs)
                    g[j] = pltpu.async_copy(msg_ref.at[sds[cs].at[0]],
                                            rows[b], gsems[b])
                for j in range(2):
                    g[j].wait()
                    pltpu.async_copy(rows[j], aggsh.at[sds[2 * h + j].at[1]],
                                     ssems[j], add=True)
            return carry

        lax.fori_loop(0, NQ, ebody, 0)
        sadd_wait(0, 2)
        sadd_wait(1, 3)

    @pl.when(c == 0)
    def _():
        run_edges(msga)

    @pl.when(c == 1)
    def _():
        run_edges(msgb)

    plsc.subcore_barrier()

    def make_obody(out_ref):
        def obody(i, carry):
            off = (s + i * NTILES) * ZROWS
            pltpu.sync_copy(aggsh.at[pl.ds(off, ZROWS)],
                            out_ref.at[pl.ds(off, ZROWS)])
            return carry
        return obody

    @pl.when(c == 0)
    def _():
        lax.fori_loop(0, nz, make_obody(agga), 0)

    @pl.when(c == 1)
    def _():
        lax.fori_loop(0, nz, make_obody(aggb), 0)


@functools.cache
def _make_sc_agg():
    return pl.kernel(
        _sc_body,
        out_type=[jax.ShapeDtypeStruct((N, DHP), _f32)] * 2,
        mesh=plsc.VectorSubcoreMesh(core_axis_name="c", subcore_axis_name="s"),
        compiler_params=pltpu.CompilerParams(use_tc_tiling_on_sc=False),
        scratch_types=[
            pltpu.VMEM((2, CHUNK), jnp.int32),
            pltpu.VMEM((2, CHUNK), jnp.int32),
            pltpu.VMEM((2, CHUNK), jnp.int32),
            pltpu.VMEM((2, CHUNK), jnp.int32),
            pltpu.VMEM((CHUNK, DHP), _f32),
            pltpu.VMEM((CHUNK, DHP), _f32),
            pltpu.VMEM_SHARED((NPAD, DHP), _f32),
        ] + [pltpu.SemaphoreType.DMA] * 8,
    )


def _split_out(Wc, bc):
    WT = Wc.T
    wa = jnp.pad(WT[:, :DHH], ((0, 0), (0, DHP - DHH)))
    wb = jnp.pad(WT[:, DHH:], ((0, 0), (0, DHP - DHH)))
    ba = jnp.pad(bc[:DHH], (0, DHP - DHH)).reshape(1, DHP)
    bb = jnp.pad(bc[DHH:], (0, DHP - DHH)).reshape(1, DHP)
    return wa, ba, wb, bb


def _split_in(Wf):
    WT = Wf.T
    wa = jnp.pad(WT[:DHH], ((0, DHP - DHH), (0, 0)))
    wb = jnp.pad(WT[DHH:], ((0, DHP - DHH), (0, 0)))
    return wa, wb


@jax.jit
def kernel(x, edge_index, graph_ids, W_lift, b_lift,
           Wc1, bc1, Wf1, bf1, Wc2, bc2, Wf2, bf2, Wc3, bc3, Wf3, bf3,
           W_ro, b_ro):
    npad_e = EPAD - E
    src1 = jnp.concatenate([edge_index[0], jnp.zeros((npad_e,), jnp.int32)])
    dst1 = jnp.concatenate([edge_index[1], jnp.full((npad_e,), N, jnp.int32)])
    epairs = jnp.concatenate([src1.reshape(NCHUNKS, 1, CHUNK),
                              dst1.reshape(NCHUNKS, 1, CHUNK)], axis=1)
    zr = jnp.zeros((ZROWS, DHP), _f32)

    wl = W_lift.T
    bl = b_lift.reshape(1, DH)
    wc1a, bc1a, wc1b, bc1b = _split_out(Wc1, bc1)
    wc2a, bc2a, wc2b, bc2b = _split_out(Wc2, bc2)
    wc3a, bc3a, wc3b, bc3b = _split_out(Wc3, bc3)
    wf1a, wf1b = _split_in(Wf1)
    wf2a, wf2b = _split_in(Wf2)
    wf3a, wf3b = _split_in(Wf3)
    bf1r = bf1.reshape(1, DH)
    bf2r = bf2.reshape(1, DH)
    bf3r = bf3.reshape(1, DH)
    wro = W_ro.T
    bro = b_ro.reshape(1, DOUT)
    gid = graph_ids.reshape(N, 1)

    ma, mb = _lift_msg(x, wl, bl, wc1a, bc1a, wc1b, bc1b)
    aa, ab = _make_sc_agg()(ma, mb, epairs, zr)
    ma, mb = _mid(aa, ab, wf1a, wf1b, bf1r, wc2a, bc2a, wc2b, bc2b)
    aa, ab = _make_sc_agg()(ma, mb, epairs, zr)
    ma, mb = _mid(aa, ab, wf2a, wf2b, bf2r, wc3a, bc3a, wc3b, bc3b)
    aa, ab = _make_sc_agg()(ma, mb, epairs, zr)
    logits = _readout(aa, ab, wf3a, wf3b, bf3r, wro, bro, gid)
    return logits


# speedup vs baseline: 1.0791x; 1.0791x over previous
import functools

import jax
import jax.numpy as jnp
from jax import lax
from jax.experimental import pallas as pl
from jax.experimental.pallas import tpu as pltpu
from jax.experimental.pallas import tpu_sc as plsc

N = 10000
E = 320000
NG = 10
DIN = 10
DH = 300
DOUT = 31

DHH = 150
DHP = 160

R = 1000
NRB = N // R

CHUNK = 88
NTILES = 16
KPT = 228
NCHUNKS = NTILES * KPT
EPAD = NCHUNKS * CHUNK
NPAD = N + 16
ZROWS = 400
NZ = N // ZROWS

_f32 = jnp.float32


def _lift_msg_body(x_ref, wl_ref, bl_ref, wa_ref, ba_ref, wb_ref, bb_ref,
                   ma_ref, mb_ref):
    i = pl.program_id(0)
    h = jnp.dot(x_ref[pl.ds(i * R, R), :], wl_ref[...],
                preferred_element_type=_f32)
    h = h + bl_ref[...]
    ma = jnp.dot(h, wa_ref[...], preferred_element_type=_f32) + ba_ref[...]
    mb = jnp.dot(h, wb_ref[...], preferred_element_type=_f32) + bb_ref[...]
    ma_ref[...] = jnp.maximum(ma, 0.0)
    mb_ref[...] = jnp.maximum(mb, 0.0)


def _mid_body(aa_ref, ab_ref, wfa_ref, wfb_ref, bf_ref,
              wca_ref, bca_ref, wcb_ref, bcb_ref, ma_ref, mb_ref):
    h = (jnp.dot(aa_ref[...], wfa_ref[...], preferred_element_type=_f32)
         + jnp.dot(ab_ref[...], wfb_ref[...], preferred_element_type=_f32)
         + bf_ref[...])
    h = jnp.maximum(h, 0.0)
    ma = jnp.dot(h, wca_ref[...], preferred_element_type=_f32) + bca_ref[...]
    mb = jnp.dot(h, wcb_ref[...], preferred_element_type=_f32) + bcb_ref[...]
    ma_ref[...] = jnp.maximum(ma, 0.0)
    mb_ref[...] = jnp.maximum(mb, 0.0)


def _readout_body(aa_ref, ab_ref, wfa_ref, wfb_ref, bf_ref,
                  wro_ref, bro_ref, gid_ref, out_ref):
    i = pl.program_id(0)
    h = (jnp.dot(aa_ref[...], wfa_ref[...], preferred_element_type=_f32)
         + jnp.dot(ab_ref[...], wfb_ref[...], preferred_element_type=_f32)
         + bf_ref[...])
    h = jnp.maximum(h, 0.0)
    nl = jnp.dot(h, wro_ref[...], preferred_element_type=_f32) + bro_ref[...]
    gid = gid_ref[pl.ds(i * R, R), :]
    iota = lax.broadcasted_iota(jnp.int32, (R, NG), 1)
    oh = (gid == iota).astype(_f32)
    contrib = lax.dot_general(oh, nl, (((0,), (0,)), ((), ())),
                              preferred_element_type=_f32)

    @pl.when(i == 0)
    def _():
        out_ref[...] = jnp.zeros_like(out_ref)

    out_ref[...] += contrib


def _full(shape):
    return pl.BlockSpec(shape, lambda i: (0,) * len(shape))


def _rows(w):
    return pl.BlockSpec((R, w), lambda i: (i, 0))


_lift_msg = pl.pallas_call(
    _lift_msg_body,
    grid=(NRB,),
    in_specs=[
        _full((N, DIN)),
        _full((DIN, DH)), _full((1, DH)),
        _full((DH, DHP)), _full((1, DHP)),
        _full((DH, DHP)), _full((1, DHP)),
    ],
    out_specs=[_rows(DHP), _rows(DHP)],
    out_shape=[jax.ShapeDtypeStruct((N, DHP), _f32)] * 2,
)

_mid = pl.pallas_call(
    _mid_body,
    grid=(NRB,),
    in_specs=[
        _rows(DHP), _rows(DHP),
        _full((DHP, DH)), _full((DHP, DH)), _full((1, DH)),
        _full((DH, DHP)), _full((1, DHP)),
        _full((DH, DHP)), _full((1, DHP)),
    ],
    out_specs=[_rows(DHP), _rows(DHP)],
    out_shape=[jax.ShapeDtypeStruct((N, DHP), _f32)] * 2,
)

_readout = pl.pallas_call(
    _readout_body,
    grid=(NRB,),
    in_specs=[
        _rows(DHP), _rows(DHP),
        _full((DHP, DH)), _full((DHP, DH)), _full((1, DH)),
        _full((DH, DOUT)), _full((1, DOUT)),
        _full((N, 1)),
    ],
    out_specs=_full((NG, DOUT)),
    out_shape=jax.ShapeDtypeStruct((NG, DOUT), _f32),
)


def _sc_body(msga, msgb, epairs, zr, agga, aggb,
             sd0, sd1, rows0, rows1, aggsh,
             gsem0, gsem1, ssem0, ssem1):
    c = lax.axis_index("c")
    s = lax.axis_index("s")

    nz = (NZ - 1 - s) // NTILES + 1

    def zbody(i, carry):
        off = (s + i * NTILES) * ZROWS
        pltpu.sync_copy(zr, aggsh.at[pl.ds(off, ZROWS)])
        return carry

    lax.fori_loop(0, nz, zbody, 0)

    plsc.subcore_barrier()

    base = s * KPT

    def run_edges(msg_ref):
        def ebody(p, carry):
            k = 2 * p

            @pl.when(p > 0)
            def _():
                pltpu.make_async_copy(rows0, aggsh.at[sd0.at[1]],
                                      ssem0).wait()

            pltpu.sync_copy(epairs.at[base + k], sd0)
            g0 = pltpu.async_copy(msg_ref.at[sd0.at[0]], rows0, gsem0)

            @pl.when(p > 0)
            def _():
                pltpu.make_async_copy(rows1, aggsh.at[sd1.at[1]],
                                      ssem1).wait()

            pltpu.sync_copy(epairs.at[base + k + 1], sd1)
            g1 = pltpu.async_copy(msg_ref.at[sd1.at[0]], rows1, gsem1)
            g0.wait()
            pltpu.async_copy(rows0, aggsh.at[sd0.at[1]], ssem0, add=True)
            g1.wait()
            pltpu.async_copy(rows1, aggsh.at[sd1.at[1]], ssem1, add=True)
            return carry

        lax.fori_loop(0, KPT // 2, ebody, 0)
        pltpu.make_async_copy(rows0, aggsh.at[sd0.at[1]], ssem0).wait()
        pltpu.make_async_copy(rows1, aggsh.at[sd1.at[1]], ssem1).wait()

    @pl.when(c == 0)
    def _():
        run_edges(msga)

    @pl.when(c == 1)
    def _():
        run_edges(msgb)

    plsc.subcore_barrier()

    def make_obody(out_ref):
        def obody(i, carry):
            off = (s + i * NTILES) * ZROWS
            pltpu.sync_copy(aggsh.at[pl.ds(off, ZROWS)],
                            out_ref.at[pl.ds(off, ZROWS)])
            return carry
        return obody

    @pl.when(c == 0)
    def _():
        lax.fori_loop(0, nz, make_obody(agga), 0)

    @pl.when(c == 1)
    def _():
        lax.fori_loop(0, nz, make_obody(aggb), 0)


@functools.cache
def _make_sc_agg():
    return pl.kernel(
        _sc_body,
        out_type=[jax.ShapeDtypeStruct((N, DHP), _f32)] * 2,
        mesh=plsc.VectorSubcoreMesh(core_axis_name="c", subcore_axis_name="s"),
        compiler_params=pltpu.CompilerParams(use_tc_tiling_on_sc=False),
        scratch_types=[
            pltpu.VMEM((2, CHUNK), jnp.int32),
            pltpu.VMEM((2, CHUNK), jnp.int32),
            pltpu.VMEM((CHUNK, DHP), _f32),
            pltpu.VMEM((CHUNK, DHP), _f32),
            pltpu.VMEM_SHARED((NPAD, DHP), _f32),
            pltpu.SemaphoreType.DMA,
            pltpu.SemaphoreType.DMA,
            pltpu.SemaphoreType.DMA,
            pltpu.SemaphoreType.DMA,
        ],
    )


def _split_out(Wc, bc):
    WT = Wc.T
    wa = jnp.pad(WT[:, :DHH], ((0, 0), (0, DHP - DHH)))
    wb = jnp.pad(WT[:, DHH:], ((0, 0), (0, DHP - DHH)))
    ba = jnp.pad(bc[:DHH], (0, DHP - DHH)).reshape(1, DHP)
    bb = jnp.pad(bc[DHH:], (0, DHP - DHH)).reshape(1, DHP)
    return wa, ba, wb, bb


def _split_in(Wf):
    WT = Wf.T
    wa = jnp.pad(WT[:DHH], ((0, DHP - DHH), (0, 0)))
    wb = jnp.pad(WT[DHH:], ((0, DHP - DHH), (0, 0)))
    return wa, wb


@jax.jit
def kernel(x, edge_index, graph_ids, W_lift, b_lift,
           Wc1, bc1, Wf1, bf1, Wc2, bc2, Wf2, bf2, Wc3, bc3, Wf3, bf3,
           W_ro, b_ro):
    npad_e = EPAD - E
    src1 = jnp.concatenate([edge_index[0], jnp.zeros((npad_e,), jnp.int32)])
    dst1 = jnp.concatenate([edge_index[1], jnp.full((npad_e,), N, jnp.int32)])
    epairs = jnp.concatenate([src1.reshape(NCHUNKS, 1, CHUNK),
                              dst1.reshape(NCHUNKS, 1, CHUNK)], axis=1)
    zr = jnp.zeros((ZROWS, DHP), _f32)

    wl = W_lift.T
    bl = b_lift.reshape(1, DH)
    wc1a, bc1a, wc1b, bc1b = _split_out(Wc1, bc1)
    wc2a, bc2a, wc2b, bc2b = _split_out(Wc2, bc2)
    wc3a, bc3a, wc3b, bc3b = _split_out(Wc3, bc3)
    wf1a, wf1b = _split_in(Wf1)
    wf2a, wf2b = _split_in(Wf2)
    wf3a, wf3b = _split_in(Wf3)
    bf1r = bf1.reshape(1, DH)
    bf2r = bf2.reshape(1, DH)
    bf3r = bf3.reshape(1, DH)
    wro = W_ro.T
    bro = b_ro.reshape(1, DOUT)
    gid = graph_ids.reshape(N, 1)

    ma, mb = _lift_msg(x, wl, bl, wc1a, bc1a, wc1b, bc1b)
    aa, ab = _make_sc_agg()(ma, mb, epairs, zr)
    ma, mb = _mid(aa, ab, wf1a, wf1b, bf1r, wc2a, bc2a, wc2b, bc2b)
    aa, ab = _make_sc_agg()(ma, mb, epairs, zr)
    ma, mb = _mid(aa, ab, wf2a, wf2b, bf2r, wc3a, bc3a, wc3b, bc3b)
    aa, ab = _make_sc_agg()(ma, mb, epairs, zr)
    logits = _readout(aa, ab, wf3a, wf3b, bf3r, wro, bro, gid)
    return logits


# speedup vs baseline: 1.1254x; 1.0429x over previous
import functools

import jax
import jax.numpy as jnp
from jax import lax
from jax.experimental import pallas as pl
from jax.experimental.pallas import tpu as pltpu
from jax.experimental.pallas import tpu_sc as plsc

N = 10000
E = 320000
NG = 10
DIN = 10
DH = 300
DOUT = 31

DHH = 150
DHP = 160

R = 2000
NRB = N // R

CHUNK = 92
NTILES = 16
KPT = 218
NCHUNKS = NTILES * KPT
EPAD = NCHUNKS * CHUNK
NPAD = N + 16
ZROWS = 400
NZ = N // ZROWS

_f32 = jnp.float32


def _lift_msg_body(x_ref, wl_ref, bl_ref, wa_ref, ba_ref, wb_ref, bb_ref,
                   ma_ref, mb_ref):
    i = pl.program_id(0)
    h = jnp.dot(x_ref[pl.ds(i * R, R), :], wl_ref[...],
                preferred_element_type=_f32)
    h = h + bl_ref[...]
    ma = jnp.dot(h, wa_ref[...], preferred_element_type=_f32) + ba_ref[...]
    mb = jnp.dot(h, wb_ref[...], preferred_element_type=_f32) + bb_ref[...]
    ma_ref[...] = jnp.maximum(ma, 0.0)
    mb_ref[...] = jnp.maximum(mb, 0.0)


def _mid_body(aa_ref, ab_ref, wfa_ref, wfb_ref, bf_ref,
              wca_ref, bca_ref, wcb_ref, bcb_ref, ma_ref, mb_ref):
    h = (jnp.dot(aa_ref[...], wfa_ref[...], preferred_element_type=_f32)
         + jnp.dot(ab_ref[...], wfb_ref[...], preferred_element_type=_f32)
         + bf_ref[...])
    h = jnp.maximum(h, 0.0)
    ma = jnp.dot(h, wca_ref[...], preferred_element_type=_f32) + bca_ref[...]
    mb = jnp.dot(h, wcb_ref[...], preferred_element_type=_f32) + bcb_ref[...]
    ma_ref[...] = jnp.maximum(ma, 0.0)
    mb_ref[...] = jnp.maximum(mb, 0.0)


def _readout_body(aa_ref, ab_ref, wfa_ref, wfb_ref, bf_ref,
                  wro_ref, bro_ref, gid_ref, out_ref):
    i = pl.program_id(0)
    h = (jnp.dot(aa_ref[...], wfa_ref[...], preferred_element_type=_f32)
         + jnp.dot(ab_ref[...], wfb_ref[...], preferred_element_type=_f32)
         + bf_ref[...])
    h = jnp.maximum(h, 0.0)
    nl = jnp.dot(h, wro_ref[...], preferred_element_type=_f32) + bro_ref[...]
    gid = gid_ref[pl.ds(i * R, R), :]
    iota = lax.broadcasted_iota(jnp.int32, (R, NG), 1)
    oh = (gid == iota).astype(_f32)
    contrib = lax.dot_general(oh, nl, (((0,), (0,)), ((), ())),
                              preferred_element_type=_f32)

    @pl.when(i == 0)
    def _():
        out_ref[...] = jnp.zeros_like(out_ref)

    out_ref[...] += contrib


def _full(shape):
    return pl.BlockSpec(shape, lambda i: (0,) * len(shape))


def _rows(w):
    return pl.BlockSpec((R, w), lambda i: (i, 0))


_lift_msg = pl.pallas_call(
    _lift_msg_body,
    grid=(NRB,),
    in_specs=[
        _full((N, DIN)),
        _full((DIN, DH)), _full((1, DH)),
        _full((DH, DHP)), _full((1, DHP)),
        _full((DH, DHP)), _full((1, DHP)),
    ],
    out_specs=[_rows(DHP), _rows(DHP)],
    out_shape=[jax.ShapeDtypeStruct((N, DHP), _f32)] * 2,
)

_mid = pl.pallas_call(
    _mid_body,
    grid=(NRB,),
    in_specs=[
        _rows(DHP), _rows(DHP),
        _full((DHP, DH)), _full((DHP, DH)), _full((1, DH)),
        _full((DH, DHP)), _full((1, DHP)),
        _full((DH, DHP)), _full((1, DHP)),
    ],
    out_specs=[_rows(DHP), _rows(DHP)],
    out_shape=[jax.ShapeDtypeStruct((N, DHP), _f32)] * 2,
)

_readout = pl.pallas_call(
    _readout_body,
    grid=(NRB,),
    in_specs=[
        _rows(DHP), _rows(DHP),
        _full((DHP, DH)), _full((DHP, DH)), _full((1, DH)),
        _full((DH, DOUT)), _full((1, DOUT)),
        _full((N, 1)),
    ],
    out_specs=_full((NG, DOUT)),
    out_shape=jax.ShapeDtypeStruct((NG, DOUT), _f32),
)


def _sc_body(msga, msgb, epairs, zr, agga, aggb,
             sd0, sd1, rows0, rows1, aggsh,
             gsem0, gsem1, ssem0, ssem1):
    c = lax.axis_index("c")
    s = lax.axis_index("s")

    nz = (NZ - 1 - s) // NTILES + 1

    def zbody(i, carry):
        off = (s + i * NTILES) * ZROWS
        pltpu.sync_copy(zr, aggsh.at[pl.ds(off, ZROWS)])
        return carry

    lax.fori_loop(0, nz, zbody, 0)

    plsc.subcore_barrier()

    base = s * KPT

    def run_edges(msg_ref):
        def ebody(p, carry):
            k = 2 * p

            @pl.when(p > 0)
            def _():
                pltpu.make_async_copy(rows0, aggsh.at[sd0.at[1]],
                                      ssem0).wait()

            pltpu.sync_copy(epairs.at[base + k], sd0)
            g0 = pltpu.async_copy(msg_ref.at[sd0.at[0]], rows0, gsem0)

            @pl.when(p > 0)
            def _():
                pltpu.make_async_copy(rows1, aggsh.at[sd1.at[1]],
                                      ssem1).wait()

            pltpu.sync_copy(epairs.at[base + k + 1], sd1)
            g1 = pltpu.async_copy(msg_ref.at[sd1.at[0]], rows1, gsem1)
            g0.wait()
            pltpu.async_copy(rows0, aggsh.at[sd0.at[1]], ssem0, add=True)
            g1.wait()
            pltpu.async_copy(rows1, aggsh.at[sd1.at[1]], ssem1, add=True)
            return carry

        lax.fori_loop(0, KPT // 2, ebody, 0)
        pltpu.make_async_copy(rows0, aggsh.at[sd0.at[1]], ssem0).wait()
        pltpu.make_async_copy(rows1, aggsh.at[sd1.at[1]], ssem1).wait()

    @pl.when(c == 0)
    def _():
        run_edges(msga)

    @pl.when(c == 1)
    def _():
        run_edges(msgb)

    plsc.subcore_barrier()

    def make_obody(out_ref):
        def obody(i, carry):
            off = (s + i * NTILES) * ZROWS
            pltpu.sync_copy(aggsh.at[pl.ds(off, ZROWS)],
                            out_ref.at[pl.ds(off, ZROWS)])
            return carry
        return obody

    @pl.when(c == 0)
    def _():
        lax.fori_loop(0, nz, make_obody(agga), 0)

    @pl.when(c == 1)
    def _():
        lax.fori_loop(0, nz, make_obody(aggb), 0)


@functools.cache
def _make_sc_agg():
    return pl.kernel(
        _sc_body,
        out_type=[jax.ShapeDtypeStruct((N, DHP), _f32)] * 2,
        mesh=plsc.VectorSubcoreMesh(core_axis_name="c", subcore_axis_name="s"),
        compiler_params=pltpu.CompilerParams(use_tc_tiling_on_sc=False),
        scratch_types=[
            pltpu.VMEM((2, CHUNK), jnp.int32),
            pltpu.VMEM((2, CHUNK), jnp.int32),
            pltpu.VMEM((CHUNK, DHP), _f32),
            pltpu.VMEM((CHUNK, DHP), _f32),
            pltpu.VMEM_SHARED((NPAD, DHP), _f32),
            pltpu.SemaphoreType.DMA,
            pltpu.SemaphoreType.DMA,
            pltpu.SemaphoreType.DMA,
            pltpu.SemaphoreType.DMA,
        ],
    )


def _split_out(Wc, bc):
    WT = Wc.T
    wa = jnp.pad(WT[:, :DHH], ((0, 0), (0, DHP - DHH)))
    wb = jnp.pad(WT[:, DHH:], ((0, 0), (0, DHP - DHH)))
    ba = jnp.pad(bc[:DHH], (0, DHP - DHH)).reshape(1, DHP)
    bb = jnp.pad(bc[DHH:], (0, DHP - DHH)).reshape(1, DHP)
    return wa, ba, wb, bb


def _split_in(Wf):
    WT = Wf.T
    wa = jnp.pad(WT[:DHH], ((0, DHP - DHH), (0, 0)))
    wb = jnp.pad(WT[DHH:], ((0, DHP - DHH), (0, 0)))
    return wa, wb


@jax.jit
def kernel(x, edge_index, graph_ids, W_lift, b_lift,
           Wc1, bc1, Wf1, bf1, Wc2, bc2, Wf2, bf2, Wc3, bc3, Wf3, bf3,
           W_ro, b_ro):
    npad_e = EPAD - E
    src1 = jnp.concatenate([edge_index[0], jnp.zeros((npad_e,), jnp.int32)])
    dst1 = jnp.concatenate([edge_index[1], jnp.full((npad_e,), N, jnp.int32)])
    epairs = jnp.concatenate([src1.reshape(NCHUNKS, 1, CHUNK),
                              dst1.reshape(NCHUNKS, 1, CHUNK)], axis=1)
    zr = jnp.zeros((ZROWS, DHP), _f32)

    wl = W_lift.T
    bl = b_lift.reshape(1, DH)
    wc1a, bc1a, wc1b, bc1b = _split_out(Wc1, bc1)
    wc2a, bc2a, wc2b, bc2b = _split_out(Wc2, bc2)
    wc3a, bc3a, wc3b, bc3b = _split_out(Wc3, bc3)
    wf1a, wf1b = _split_in(Wf1)
    wf2a, wf2b = _split_in(Wf2)
    wf3a, wf3b = _split_in(Wf3)
    bf1r = bf1.reshape(1, DH)
    bf2r = bf2.reshape(1, DH)
    bf3r = bf3.reshape(1, DH)
    wro = W_ro.T
    bro = b_ro.reshape(1, DOUT)
    gid = graph_ids.reshape(N, 1)

    ma, mb = _lift_msg(x, wl, bl, wc1a, bc1a, wc1b, bc1b)
    aa, ab = _make_sc_agg()(ma, mb, epairs, zr)
    ma, mb = _mid(aa, ab, wf1a, wf1b, bf1r, wc2a, bc2a, wc2b, bc2b)
    aa, ab = _make_sc_agg()(ma, mb, epairs, zr)
    ma, mb = _mid(aa, ab, wf2a, wf2b, bf2r, wc3a, bc3a, wc3b, bc3b)
    aa, ab = _make_sc_agg()(ma, mb, epairs, zr)
    logits = _readout(aa, ab, wf3a, wf3b, bf3r, wro, bro, gid)
    return logits
